# Initial kernel scaffold; baseline (speedup 1.0000x reference)
import jax
import jax.numpy as jnp
from jax.experimental import pallas as pl


def kernel(x, adj, enc_w1, enc_b1, enc_w2, enc_b2, gc1_w, gc2_w, gc3_w, att_w, dec_w1, dec_b1, dec_w2, dec_b2):
    raise NotImplementedError("write your pallas kernel here")


# speedup vs baseline: 1.4072x; 1.4072x over previous
import math

import jax
import jax.numpy as jnp
from jax.experimental import pallas as pl
from jax.experimental.pallas import tpu as pltpu

_BN_SCALE = 1.0 / math.sqrt(1.0 + 1e-5)
_ROW_BLOCK = 400
_VMEM_LIMIT = 56 * 1024 * 1024


def _support_kernel(x_ref, w_ref, s_ref):
    s_ref[...] = jnp.dot(x_ref[...], w_ref[...],
                         preferred_element_type=jnp.float32)


def _hop1_kernel(adj_ref, s_ref, g23_ref, t_ref):
    h1 = jnp.maximum(
        jnp.dot(adj_ref[...], s_ref[...], preferred_element_type=jnp.float32),
        0.0)
    t_ref[...] = jnp.dot(h1, g23_ref[...], preferred_element_type=jnp.float32)


def _hop2_fused_kernel(adj_ref, t_ref, x_ref, ew1_ref, eb1_ref, ew2_ref,
                       eb2_ref, att_ref, dw1_ref, db1_ref, dw2_ref, db2_ref,
                       z_ref, mu_ref, lv_ref, df_ref):
    gh2 = mu_ref.shape[1]
    ml = jnp.dot(adj_ref[...], t_ref[...], preferred_element_type=jnp.float32)
    mu = ml[:, :gh2]
    lv = ml[:, gh2:]
    h = jnp.maximum(
        (jnp.dot(x_ref[...], ew1_ref[...], preferred_element_type=jnp.float32)
         + eb1_ref[...]) * _BN_SCALE, 0.0)
    feat = jnp.maximum(
        (jnp.dot(h, ew2_ref[...], preferred_element_type=jnp.float32)
         + eb2_ref[...]) * _BN_SCALE, 0.0)
    att = att_ref[...]
    wg = jnp.sum(mu * att, axis=1, keepdims=True)
    wf = jnp.sum(feat * att, axis=1, keepdims=True)
    m = jnp.maximum(wg, wf)
    eg = jnp.exp(wg - m)
    ef = jnp.exp(wf - m)
    z = (eg * mu + ef * feat) / (eg + ef)
    dh = jnp.maximum(
        (jnp.dot(z, dw1_ref[...], preferred_element_type=jnp.float32)
         + db1_ref[...]) * _BN_SCALE, 0.0)
    df_ref[...] = (jnp.dot(dh, dw2_ref[...], preferred_element_type=jnp.float32)
                   + db2_ref[...])
    z_ref[...] = z
    mu_ref[...] = mu
    lv_ref[...] = lv


def kernel(x, adj, enc_w1, enc_b1, enc_w2, enc_b2, gc1_w, gc2_w, gc3_w,
           att_w, dec_w1, dec_b1, dec_w2, dec_b2):
    n, d = x.shape
    gh1 = gc1_w.shape[1]
    gh2 = gc2_w.shape[1]
    fh0 = enc_w1.shape[1]
    fh2 = enc_w2.shape[1]
    blk = _ROW_BLOCK
    nb = n // blk

    s = pl.pallas_call(
        _support_kernel,
        out_shape=jax.ShapeDtypeStruct((n, gh1), jnp.float32),
    )(x, gc1_w)

    g23 = jnp.concatenate([gc2_w, gc3_w], axis=1)
    t = pl.pallas_call(
        _hop1_kernel,
        grid=(nb,),
        in_specs=[
            pl.BlockSpec((blk, n), lambda i: (i, 0)),
            pl.BlockSpec((n, gh1), lambda i: (0, 0)),
            pl.BlockSpec((gh1, 2 * gh2), lambda i: (0, 0)),
        ],
        out_specs=pl.BlockSpec((blk, 2 * gh2), lambda i: (i, 0)),
        out_shape=jax.ShapeDtypeStruct((n, 2 * gh2), jnp.float32),
        compiler_params=pltpu.CompilerParams(vmem_limit_bytes=_VMEM_LIMIT),
    )(adj, s, g23)

    eb1 = enc_b1.reshape(1, fh0)
    eb2 = enc_b2.reshape(1, fh2)
    db1 = dec_b1.reshape(1, fh0)
    db2 = dec_b2.reshape(1, d)
    att = att_w.reshape(1, gh2)

    z, mu, lv, df = pl.pallas_call(
        _hop2_fused_kernel,
        grid=(nb,),
        in_specs=[
            pl.BlockSpec((blk, n), lambda i: (i, 0)),
            pl.BlockSpec((n, 2 * gh2), lambda i: (0, 0)),
            pl.BlockSpec((blk, d), lambda i: (i, 0)),
            pl.BlockSpec((d, fh0), lambda i: (0, 0)),
            pl.BlockSpec((1, fh0), lambda i: (0, 0)),
            pl.BlockSpec((fh0, fh2), lambda i: (0, 0)),
            pl.BlockSpec((1, fh2), lambda i: (0, 0)),
            pl.BlockSpec((1, gh2), lambda i: (0, 0)),
            pl.BlockSpec((fh2, fh0), lambda i: (0, 0)),
            pl.BlockSpec((1, fh0), lambda i: (0, 0)),
            pl.BlockSpec((fh0, d), lambda i: (0, 0)),
            pl.BlockSpec((1, d), lambda i: (0, 0)),
        ],
        out_specs=[
            pl.BlockSpec((blk, gh2), lambda i: (i, 0)),
            pl.BlockSpec((blk, gh2), lambda i: (i, 0)),
            pl.BlockSpec((blk, gh2), lambda i: (i, 0)),
            pl.BlockSpec((blk, d), lambda i: (i, 0)),
        ],
        out_shape=(
            jax.ShapeDtypeStruct((n, gh2), jnp.float32),
            jax.ShapeDtypeStruct((n, gh2), jnp.float32),
            jax.ShapeDtypeStruct((n, gh2), jnp.float32),
            jax.ShapeDtypeStruct((n, d), jnp.float32),
        ),
        compiler_params=pltpu.CompilerParams(vmem_limit_bytes=_VMEM_LIMIT),
    )(adj, t, x, enc_w1, eb1, enc_w2, eb2, att, dec_w1, db1, dec_w2, db2)
    return (z, mu, lv, df)
